# baseline (speedup 1.0000x reference)
import math

import jax
import jax.numpy as jnp
from jax.experimental import pallas as pl

N_Q = 8
BINS = 1024
DIM = 256
CODE_DIM = 8
NCHAINS = 2


def _rvq_kernel(x_ref, pw_ref, pb_ref, pow_ref, pob_ref, cb_ref, c2h_ref,
                cbt_ref, q_out_ref, codes_ref, loss_ref):
    Tt = x_ref.shape[2]
    H = Tt // NCHAINS

    def stage(i, r):
        P = pw_ref[i]
        xp = jax.lax.dot_general(P, r, (((1,), (0,)), ((), ())),
                                 preferred_element_type=jnp.float32)
        xp = xp + pb_ref[i][:, None]
        s = jax.lax.dot_general(cb_ref[i], xp, (((1,), (0,)), ((), ())),
                                preferred_element_type=jnp.float32)
        sc = c2h_ref[i][:, None] - s
        m = jnp.min(sc, axis=0)
        onehot = (sc <= m[None, :]).astype(jnp.float32)
        fused = jax.lax.dot_general(cbt_ref[i], onehot, (((1,), (0,)), ((), ())),
                                    preferred_element_type=jnp.float32)
        q = fused[:CODE_DIM]
        idx = fused[CODE_DIM].astype(jnp.int32)
        lp = jnp.sum((q - xp) ** 2, axis=0)
        qo = jax.lax.dot_general(pow_ref[i], q, (((1,), (0,)), ((), ())),
                                 preferred_element_type=jnp.float32)
        qo = qo + pob_ref[i][:, None]
        return r - qo, qo, idx, lp

    chains = []
    for h in range(NCHAINS):
        r = x_ref[0, :, h * H:(h + 1) * H]
        chains.append({"r": r, "qacc": jnp.zeros_like(r), "idx": [], "lp": []})

    for i in range(N_Q):
        for st in chains:
            r, qo, idx, lp = stage(i, st["r"])
            st["r"] = r
            st["qacc"] = st["qacc"] + qo
            st["idx"].append(idx)
            st["lp"].append(lp)

    for h, st in enumerate(chains):
        sl = pl.ds(h * H, H)
        q_out_ref[0, :, sl] = st["qacc"]
        codes_ref[0, :, sl] = jnp.stack(st["idx"], axis=0)
        loss_ref[0, :, sl] = jnp.stack(st["lp"], axis=0)


def kernel(x, frame_rate, proj_in_w, proj_in_b, proj_out_w, proj_out_b, codebooks):
    B, D, T = x.shape
    Tt = 2048
    grid = (B, T // Tt)

    c2h = 0.5 * jnp.sum(codebooks * codebooks, axis=-1)
    iota_row = jnp.arange(BINS, dtype=jnp.float32)[None, None, :]
    cbt = jnp.concatenate(
        [jnp.transpose(codebooks, (0, 2, 1)),
         jnp.broadcast_to(iota_row, (N_Q, 1, BINS))], axis=1)

    quantized, codes_tmp, loss_parts = pl.pallas_call(
        _rvq_kernel,
        grid=grid,
        in_specs=[
            pl.BlockSpec((1, D, Tt), lambda b, t: (b, 0, t)),
            pl.BlockSpec((N_Q, CODE_DIM, D), lambda b, t: (0, 0, 0)),
            pl.BlockSpec((N_Q, CODE_DIM), lambda b, t: (0, 0)),
            pl.BlockSpec((N_Q, D, CODE_DIM), lambda b, t: (0, 0, 0)),
            pl.BlockSpec((N_Q, D), lambda b, t: (0, 0)),
            pl.BlockSpec((N_Q, BINS, CODE_DIM), lambda b, t: (0, 0, 0)),
            pl.BlockSpec((N_Q, BINS), lambda b, t: (0, 0)),
            pl.BlockSpec((N_Q, CODE_DIM + 1, BINS), lambda b, t: (0, 0, 0)),
        ],
        out_specs=[
            pl.BlockSpec((1, D, Tt), lambda b, t: (b, 0, t)),
            pl.BlockSpec((1, N_Q, Tt), lambda b, t: (b, 0, t)),
            pl.BlockSpec((1, N_Q, Tt), lambda b, t: (b, 0, t)),
        ],
        out_shape=[
            jax.ShapeDtypeStruct((B, D, T), jnp.float32),
            jax.ShapeDtypeStruct((B, N_Q, T), jnp.int32),
            jax.ShapeDtypeStruct((B, N_Q, T), jnp.float32),
        ],
    )(x, proj_in_w, proj_in_b, proj_out_w, proj_out_b, codebooks, c2h, cbt)

    codes = jnp.transpose(codes_tmp, (1, 0, 2))
    commit_loss = jnp.sum(loss_parts, axis=(0, 2)) / (B * T * CODE_DIM)
    bw = jnp.asarray(N_Q * math.log2(BINS) * frame_rate, x.dtype)
    return quantized, codes, bw, commit_loss


# speedup vs baseline: 1.2956x; 1.2956x over previous
import math

import jax
import jax.numpy as jnp
from jax.experimental import pallas as pl

N_Q = 8
BINS = 1024
DIM = 256
CODE_DIM = 8
NCHAINS = 1


def _rvq_kernel(x_ref, pw_ref, pb_ref, pow_ref, pob_ref, cb_ref, c2h_ref,
                q_out_ref, codes_ref, loss_ref):
    Tt = x_ref.shape[2]
    H = Tt // NCHAINS
    row_iota = jax.lax.broadcasted_iota(jnp.int32, (BINS, H), 0)

    def stage(i, r):
        P = pw_ref[i]
        xp = jax.lax.dot_general(P, r, (((1,), (0,)), ((), ())),
                                 preferred_element_type=jnp.float32)
        xp = xp + pb_ref[i][:, None]
        s = jax.lax.dot_general(cb_ref[i], xp, (((1,), (0,)), ((), ())),
                                preferred_element_type=jnp.float32)
        sc = c2h_ref[i][:, None] - s
        idx = jnp.argmin(sc, axis=0)
        onehot = (row_iota == idx[None, :]).astype(jnp.float32)
        q = jax.lax.dot_general(cb_ref[i], onehot, (((0,), (0,)), ((), ())),
                                preferred_element_type=jnp.float32)
        lp = jnp.sum((q - xp) ** 2, axis=0)
        qo = jax.lax.dot_general(pow_ref[i], q, (((1,), (0,)), ((), ())),
                                 preferred_element_type=jnp.float32)
        qo = qo + pob_ref[i][:, None]
        return r - qo, idx, lp

    chains = []
    for h in range(NCHAINS):
        r = x_ref[0, :, h * H:(h + 1) * H]
        chains.append({"r": r, "idx": [], "lp": []})

    for i in range(N_Q):
        for st in chains:
            r, idx, lp = stage(i, st["r"])
            st["r"] = r
            st["idx"].append(idx)
            st["lp"].append(lp)

    for h, st in enumerate(chains):
        sl = pl.ds(h * H, H)
        q_out_ref[0, :, sl] = x_ref[0, :, sl] - st["r"]
        codes_ref[0, :, sl] = jnp.stack(st["idx"], axis=0)
        loss_ref[0, :, sl] = jnp.stack(st["lp"], axis=0)


def kernel(x, frame_rate, proj_in_w, proj_in_b, proj_out_w, proj_out_b, codebooks):
    B, D, T = x.shape
    Tt = 2048
    grid = (B, T // Tt)

    c2h = 0.5 * jnp.sum(codebooks * codebooks, axis=-1)

    quantized, codes_tmp, loss_parts = pl.pallas_call(
        _rvq_kernel,
        grid=grid,
        in_specs=[
            pl.BlockSpec((1, D, Tt), lambda b, t: (b, 0, t)),
            pl.BlockSpec((N_Q, CODE_DIM, D), lambda b, t: (0, 0, 0)),
            pl.BlockSpec((N_Q, CODE_DIM), lambda b, t: (0, 0)),
            pl.BlockSpec((N_Q, D, CODE_DIM), lambda b, t: (0, 0, 0)),
            pl.BlockSpec((N_Q, D), lambda b, t: (0, 0)),
            pl.BlockSpec((N_Q, BINS, CODE_DIM), lambda b, t: (0, 0, 0)),
            pl.BlockSpec((N_Q, BINS), lambda b, t: (0, 0)),
        ],
        out_specs=[
            pl.BlockSpec((1, D, Tt), lambda b, t: (b, 0, t)),
            pl.BlockSpec((1, N_Q, Tt), lambda b, t: (b, 0, t)),
            pl.BlockSpec((1, N_Q, Tt), lambda b, t: (b, 0, t)),
        ],
        out_shape=[
            jax.ShapeDtypeStruct((B, D, T), jnp.float32),
            jax.ShapeDtypeStruct((B, N_Q, T), jnp.int32),
            jax.ShapeDtypeStruct((B, N_Q, T), jnp.float32),
        ],
    )(x, proj_in_w, proj_in_b, proj_out_w, proj_out_b, codebooks, c2h)

    codes = jnp.transpose(codes_tmp, (1, 0, 2))
    commit_loss = jnp.sum(loss_parts, axis=(0, 2)) / (B * T * CODE_DIM)
    bw = jnp.asarray(N_Q * math.log2(BINS) * frame_rate, x.dtype)
    return quantized, codes, bw, commit_loss
